# baseline (speedup 1.0000x reference)
import functools

import jax
import jax.numpy as jnp
from jax import lax
from jax.experimental import pallas as pl
from jax.experimental.pallas import tpu as pltpu
from jax.experimental.pallas import tpu_sc as plsc

MAX_SEQ_LEN = 8192
EMBED_DIM = 1024

_NUM_CORES = 2
_NUM_SUBCORES = 16
_NUM_WORKERS = _NUM_CORES * _NUM_SUBCORES
_ROWS_PER_WORKER = MAX_SEQ_LEN // _NUM_WORKERS

_mesh = plsc.VectorSubcoreMesh(core_axis_name="c", subcore_axis_name="s")


@functools.partial(
    pl.kernel,
    mesh=_mesh,
    out_type=jax.ShapeDtypeStruct((MAX_SEQ_LEN, EMBED_DIM), jnp.float32),
)
def _copy_table(table_hbm, out_hbm):
    wid = lax.axis_index("s") * _NUM_CORES + lax.axis_index("c")
    base = wid * _ROWS_PER_WORKER
    pltpu.sync_copy(
        table_hbm.at[pl.ds(base, _ROWS_PER_WORKER)],
        out_hbm.at[pl.ds(base, _ROWS_PER_WORKER)],
    )


def kernel(input, table):
    del input
    return _copy_table(table)[None]


# speedup vs baseline: 24.2705x; 24.2705x over previous
import functools

import jax
import jax.numpy as jnp
from jax import lax
from jax.experimental import pallas as pl
from jax.experimental.pallas import tpu as pltpu
from jax.experimental.pallas import tpu_sc as plsc

MAX_SEQ_LEN = 8192
EMBED_DIM = 1024

_NUM_CORES = 2
_NUM_SUBCORES = 16
_NUM_WORKERS = _NUM_CORES * _NUM_SUBCORES
_ROWS_PER_WORKER = MAX_SEQ_LEN // _NUM_WORKERS

_CHUNK = 32
_NCHUNK = _ROWS_PER_WORKER // _CHUNK

_mesh = plsc.VectorSubcoreMesh(core_axis_name="c", subcore_axis_name="s")


@functools.partial(
    pl.kernel,
    mesh=_mesh,
    out_type=jax.ShapeDtypeStruct((MAX_SEQ_LEN, EMBED_DIM), jnp.float32),
    scratch_types=[
        pltpu.VMEM((_CHUNK, EMBED_DIM), jnp.float32),
        pltpu.VMEM((_CHUNK, EMBED_DIM), jnp.float32),
        pltpu.SemaphoreType.DMA,
        pltpu.SemaphoreType.DMA,
        pltpu.SemaphoreType.DMA,
        pltpu.SemaphoreType.DMA,
    ],
)
def _copy_table(table_hbm, out_hbm, buf0, buf1, ls0, ls1, ss0, ss1):
    wid = lax.axis_index("s") * _NUM_CORES + lax.axis_index("c")
    base = wid * _ROWS_PER_WORKER
    bufs = (buf0, buf1)
    lsems = (ls0, ls1)
    ssems = (ss0, ss1)

    def load(g):
        return pltpu.async_copy(
            table_hbm.at[pl.ds(base + g * _CHUNK, _CHUNK)],
            bufs[g % 2],
            lsems[g % 2],
        )

    def store(g):
        return pltpu.async_copy(
            bufs[g % 2],
            out_hbm.at[pl.ds(base + g * _CHUNK, _CHUNK)],
            ssems[g % 2],
        )

    loads = [None] * _NCHUNK
    stores = [None] * _NCHUNK
    loads[0] = load(0)
    for g in range(_NCHUNK):
        if g + 1 < _NCHUNK:
            if g >= 1:
                stores[g - 1].wait()
            loads[g + 1] = load(g + 1)
        loads[g].wait()
        stores[g] = store(g)
    stores[_NCHUNK - 2].wait()
    stores[_NCHUNK - 1].wait()


def kernel(input, table):
    del input
    return _copy_table(table)[None]


# speedup vs baseline: 24.4243x; 1.0063x over previous
import functools

import jax
import jax.numpy as jnp
from jax import lax
from jax.experimental import pallas as pl
from jax.experimental.pallas import tpu as pltpu
from jax.experimental.pallas import tpu_sc as plsc

MAX_SEQ_LEN = 8192
EMBED_DIM = 1024

_NUM_CORES = 2
_NUM_SUBCORES = 16
_NUM_WORKERS = _NUM_CORES * _NUM_SUBCORES
_ROWS_PER_WORKER = MAX_SEQ_LEN // _NUM_WORKERS

_CHUNK = 16
_NCHUNK = _ROWS_PER_WORKER // _CHUNK
_NBUF = 4

_mesh = plsc.VectorSubcoreMesh(core_axis_name="c", subcore_axis_name="s")


@functools.partial(
    pl.kernel,
    mesh=_mesh,
    out_type=jax.ShapeDtypeStruct((MAX_SEQ_LEN, EMBED_DIM), jnp.float32),
    scratch_types=(
        [pltpu.VMEM((_CHUNK, EMBED_DIM), jnp.float32)] * _NBUF
        + [pltpu.SemaphoreType.DMA] * (2 * _NBUF)
    ),
)
def _copy_table(table_hbm, out_hbm, *scratch):
    bufs = scratch[:_NBUF]
    lsems = scratch[_NBUF:2 * _NBUF]
    ssems = scratch[2 * _NBUF:]
    wid = lax.axis_index("s") * _NUM_CORES + lax.axis_index("c")
    base = wid * _ROWS_PER_WORKER

    def load(g):
        return pltpu.async_copy(
            table_hbm.at[pl.ds(base + g * _CHUNK, _CHUNK)],
            bufs[g % _NBUF],
            lsems[g % _NBUF],
        )

    def store(g):
        return pltpu.async_copy(
            bufs[g % _NBUF],
            out_hbm.at[pl.ds(base + g * _CHUNK, _CHUNK)],
            ssems[g % _NBUF],
        )

    loads = [None] * _NCHUNK
    stores = [None] * _NCHUNK
    for g in range(_NBUF - 1):
        loads[g] = load(g)
    for g in range(_NCHUNK):
        n = g + _NBUF - 1
        if n < _NCHUNK:
            if g >= 1:
                stores[g - 1].wait()
            loads[n] = load(n)
        loads[g].wait()
        stores[g] = store(g)
    for g in range(max(0, _NCHUNK - _NBUF), _NCHUNK):
        stores[g].wait()


def kernel(input, table):
    del input
    return _copy_table(table)[None]
